# baseline (speedup 1.0000x reference)
import functools
import math

import jax
import jax.numpy as jnp
from jax import lax
from jax.experimental import pallas as pl
from jax.experimental.pallas import tpu as pltpu
from jax.experimental.pallas import tpu_sc as plsc

_S = 50
_D = 64
_K = 20
_BB = 32
_LN_DECAY = float(math.log(0.95))
_THRESH = 0.1
_W = 114
_ROWS = 52
_PAIR = _ROWS * 128
_NEG = -3.0e8
_NEGBIG = -3.4e38
_POISON = -1.0e9
_OUTW = _K


def _dense_body(params_ref, emb_ref, ts_ref, iw_ref, out_ref):
    lane = lax.broadcasted_iota(jnp.int32, (_S, 128), 1)
    row = lax.broadcasted_iota(jnp.int32, (_S, 128), 0)
    m0 = lane < 64
    jin = jnp.where(m0, lane, lane - 64)
    triu = (jin > row) & (jin < _S)
    linf = (row * _S + jin).astype(jnp.float32)
    m0w = m0[:, :_W]
    m0f = jnp.where(m0w[:1], 1.0, 0.0)
    zc = jnp.zeros((_S, 64), jnp.float32)
    zr = jnp.zeros((14, 128), jnp.float32)
    zpad = jnp.full((1, 14), _NEG, jnp.float32)
    for p in range(_BB // 2):
        b0, b1 = 2 * p, 2 * p + 1
        E0 = emb_ref[b0]
        E1 = emb_ref[b1]
        padE0 = jnp.concatenate([E0, zc], axis=1)
        shE1 = jnp.concatenate([zc, E1], axis=1)
        L = padE0 + shE1
        Rr = jnp.concatenate([padE0, zr, shE1], axis=0)
        S2 = lax.dot_general(L, Rr, (((1,), (1,)), ((), ())),
                             preferred_element_type=jnp.float32)
        sq = S2 * S2
        ss0 = jnp.sum(jnp.where(m0w, sq, 0.0), axis=1, keepdims=True)
        ss1 = jnp.sum(jnp.where(m0w, 0.0, sq), axis=1, keepdims=True)
        d2 = jnp.where(m0w, ss0, ss1)
        Sn = S2 / jnp.maximum(jnp.sqrt(d2), 1e-12)
        tr0 = ts_ref[b0][None, :]
        tr1 = ts_ref[b1][None, :]
        tw0 = jnp.exp((jnp.max(tr0) - tr0) * _LN_DECAY)
        tw1 = jnp.exp((jnp.max(tr1) - tr1) * _LN_DECAY)
        A = jnp.concatenate([tw0, tw1], axis=0)
        Bv = jnp.concatenate(
            [jnp.concatenate([tw0, jnp.zeros((1, 64), jnp.float32)], axis=1),
             jnp.concatenate([jnp.zeros((1, 64), jnp.float32), tw1], axis=1)],
            axis=0)[:, :_W]
        T = lax.dot_general(A, Bv, (((0,), (0,)), ((), ())),
                            preferred_element_type=jnp.float32)
        iwA = jnp.concatenate([iw_ref[b0][None, :], iw_ref[b1][None, :]],
                              axis=0)
        onesB = jnp.concatenate([m0f, 1.0 - m0f], axis=0)
        Iw = lax.dot_general(iwA, onesB, (((0,), (0,)), ((), ())),
                             preferred_element_type=jnp.float32)
        acc = jnp.full((_S, _W), params_ref[5, 0], jnp.float32)
        for c in range(16):
            a_ = params_ref[0, c]
            b_ = params_ref[1, c]
            c_ = params_ref[2, c]
            d_ = params_ref[3, c]
            e_ = params_ref[4, c]
            h = jnp.maximum(a_ * Sn + b_ * T + c_ * Iw + d_, 0.0)
            acc = acc + e_ * h
        Z = jnp.concatenate(
            [1.0 / (1.0 + jnp.exp(-acc)), jnp.zeros((_S, 128 - _W))], axis=1)
        keys = jnp.where(triu & (Z > _THRESH), Z,
                         jnp.where(triu, -10000.0 - linf, _NEG))
        r0 = jnp.max(jnp.where(m0, keys, _NEGBIG), axis=1, keepdims=True)
        r1 = jnp.max(jnp.where(m0, _NEGBIG, keys), axis=1, keepdims=True)
        rmT = jnp.concatenate([r0, r1], axis=1).T
        rm = jnp.concatenate([rmT[0:1], zpad, rmT[1:2], zpad], axis=1)
        out_ref[p, 0:_S] = Z
        out_ref[p, _S:_S + 1] = rm
        out_ref[p, _S + 1:_ROWS] = rm


def _dense_weights(item_embeddings, timestamps, interaction_weights,
                   W1, b1, W2, b2, interpret=False):
    B = item_embeddings.shape[0]
    params = jnp.stack([W1[:, 0], W1[:, 1], W1[:, 2], b1, W2[0, :],
                        jnp.broadcast_to(b2, (16,))], axis=0)
    grid = (B // _BB,)
    return pl.pallas_call(
        _dense_body,
        grid=grid,
        in_specs=[
            pl.BlockSpec((6, 16), lambda i: (0, 0), memory_space=pltpu.SMEM),
            pl.BlockSpec((_BB, _S, _D), lambda i: (i, 0, 0)),
            pl.BlockSpec((_BB, _S), lambda i: (i, 0)),
            pl.BlockSpec((_BB, _S), lambda i: (i, 0)),
        ],
        out_specs=pl.BlockSpec((_BB // 2, _ROWS, 128), lambda i: (i, 0, 0)),
        out_shape=jax.ShapeDtypeStruct((B // 2, _ROWS, 128), jnp.float32),
        interpret=interpret,
    )(params, item_embeddings, timestamps, interaction_weights)


def _sc_topk_body(nw, pairs_per_w, pbase, w_hbm, src_hbm, dst_hbm, wout_hbm,
                  row_v, osrc_v, odst_v, ow_v):
    info = plsc.get_sparse_core_info()
    nc = info.num_cores
    wid = lax.axis_index("s") * nc + lax.axis_index("c")
    iota = lax.iota(jnp.int32, 16)
    lane0 = iota == 0

    def per_pair(pp, _):
        gp = wid * pairs_per_w + pp
        pltpu.sync_copy(w_hbm.at[gp], row_v)
        for h in (0, 1):
            boff = (2 * (gp + pbase) + h) * _S
            rmbase = _S * 128 + 64 * h
            obase = (pp * 2 + h) * _OUTW

            def kstep(k, _):
                best_v = jnp.full((16,), _NEGBIG, jnp.float32)
                best_i = jnp.zeros((16,), jnp.int32)
                for q in range(4):
                    v = row_v[pl.ds(rmbase + 16 * q, 16)]
                    ridx = iota + 16 * q
                    take = v > best_v
                    best_v = jnp.where(take, v, best_v)
                    best_i = jnp.where(take, ridx, best_i)
                m = jnp.max(best_v)
                istar = jnp.min(jnp.where(best_v == m, best_i, 9999))
                base = istar * 128 + 64 * h
                keyqs = []
                cand = jnp.full((16,), 9999, jnp.int32)
                for q in range(4):
                    wv = row_v[pl.ds(base + 16 * q, 16)]
                    gj = iota + 16 * q
                    triu = (gj > istar) & (gj < _S)
                    valid = triu & (wv > _THRESH)
                    fb = triu & (wv > -1.0e8) & (~valid)
                    linf = (istar * _S + gj).astype(jnp.float32)
                    keyq = jnp.where(valid, wv,
                                     jnp.where(fb, -10000.0 - linf, _NEG))
                    keyqs.append((keyq, gj))
                    cand = jnp.minimum(cand, jnp.where(keyq == m, gj, 9999))
                jstar = jnp.min(cand)
                g = base + jstar
                gvec = jnp.full((16,), g, jnp.int32)
                raw = plsc.load_gather(row_v, [gvec])
                kvec = jnp.full((16,), obase + k, jnp.int32)
                plsc.store_scatter(ow_v, [kvec], raw, mask=lane0)
                plsc.store_scatter(osrc_v, [kvec],
                                   jnp.full((16,), istar + boff, jnp.int32),
                                   mask=lane0)
                plsc.store_scatter(odst_v, [kvec],
                                   jnp.full((16,), jstar + boff, jnp.int32),
                                   mask=lane0)
                plsc.store_scatter(row_v, [gvec],
                                   jnp.full((16,), _POISON, jnp.float32),
                                   mask=lane0)
                m3 = jnp.full((16,), _NEGBIG, jnp.float32)
                for keyq, gj in keyqs:
                    m3 = jnp.maximum(m3, jnp.where(gj == jstar, _NEG, keyq))
                plsc.store_scatter(row_v,
                                   [jnp.full((16,), rmbase + istar, jnp.int32)],
                                   jnp.full((16,), jnp.max(m3), jnp.float32),
                                   mask=lane0)
                return 0

            lax.fori_loop(0, _K, kstep, 0)
        return 0

    lax.fori_loop(0, pairs_per_w, per_pair, 0)
    pltpu.sync_copy(osrc_v, src_hbm.at[wid])
    pltpu.sync_copy(odst_v, dst_hbm.at[wid])
    pltpu.sync_copy(ow_v, wout_hbm.at[wid])


def _sc_topk(w_flat, pbase=0):
    npair = w_flat.shape[0]
    info = plsc.get_sparse_core_info()
    nw = info.num_cores * info.num_subcores
    pairs_per_w = npair // nw
    owid = pairs_per_w * 2 * _OUTW
    mesh = plsc.VectorSubcoreMesh(core_axis_name="c", subcore_axis_name="s")
    fn = functools.partial(
        pl.kernel,
        mesh=mesh,
        compiler_params=pltpu.CompilerParams(needs_layout_passes=False),
        out_type=[
            jax.ShapeDtypeStruct((nw, owid), jnp.int32),
            jax.ShapeDtypeStruct((nw, owid), jnp.int32),
            jax.ShapeDtypeStruct((nw, owid), jnp.float32),
        ],
        scratch_types=[
            pltpu.VMEM((_PAIR,), jnp.float32),
            pltpu.VMEM((owid,), jnp.int32),
            pltpu.VMEM((owid,), jnp.int32),
            pltpu.VMEM((owid,), jnp.float32),
        ],
    )(functools.partial(_sc_topk_body, nw, pairs_per_w, pbase))
    return fn(w_flat)


def kernel(user_embeddings, item_embeddings, timestamps, interaction_weights,
           W1, b1, W2, b2):
    B = item_embeddings.shape[0]
    nsplit = 4
    Bh = B // nsplit
    parts = []
    for c in range(nsplit):
        sl = slice(c * Bh, (c + 1) * Bh)
        w = _dense_weights(item_embeddings[sl], timestamps[sl],
                           interaction_weights[sl], W1, b1, W2, b2)
        parts.append(_sc_topk(w.reshape(Bh // 2, _PAIR),
                              pbase=c * (Bh // 2)))
    src = jnp.concatenate([p[0] for p in parts], axis=0)
    dst = jnp.concatenate([p[1] for p in parts], axis=0)
    edge_w = jnp.concatenate([p[2] for p in parts], axis=0)
    src = src.reshape(B, _OUTW)
    dst = dst.reshape(B, _OUTW)
    edge_w = edge_w.reshape(B, _OUTW)
    edge_index = jnp.stack([src[:, :_K].reshape(-1), dst[:, :_K].reshape(-1)],
                           axis=0)
    edge_weights = edge_w[:, :_K].reshape(-1)
    return edge_index, edge_weights


# speedup vs baseline: 1.3892x; 1.3892x over previous
import functools
import math

import jax
import jax.numpy as jnp
from jax import lax
from jax.experimental import pallas as pl
from jax.experimental.pallas import tpu as pltpu
from jax.experimental.pallas import tpu_sc as plsc

_S = 50
_D = 64
_K = 20
_BB = 16
_LN_DECAY = float(math.log(0.95))
_THRESH = 0.1
_W = 114
_ROWS = 52
_PAIR = _ROWS * 128
_NEG = -3.0e8
_NEGBIG = -3.4e38
_POISON = -1.0e9
_OUTW = _K


def _dense_body(params_ref, emb_ref, ts_ref, iw_ref, out_ref):
    lane = lax.broadcasted_iota(jnp.int32, (_S, 128), 1)
    row = lax.broadcasted_iota(jnp.int32, (_S, 128), 0)
    m0 = lane < 64
    jin = jnp.where(m0, lane, lane - 64)
    triu = (jin > row) & (jin < _S)
    linf = (row * _S + jin).astype(jnp.float32)
    m0w = m0[:, :_W]
    m0f = jnp.where(m0w[:1], 1.0, 0.0)
    zc = jnp.zeros((_S, 64), jnp.float32)
    zr = jnp.zeros((14, 128), jnp.float32)
    zpad = jnp.full((1, 14), _NEG, jnp.float32)
    for p in range(_BB // 2):
        b0, b1 = 2 * p, 2 * p + 1
        E0 = emb_ref[b0]
        E1 = emb_ref[b1]
        padE0 = jnp.concatenate([E0, zc], axis=1)
        shE1 = jnp.concatenate([zc, E1], axis=1)
        L = padE0 + shE1
        Rr = jnp.concatenate([padE0, zr, shE1], axis=0)
        S2 = lax.dot_general(L, Rr, (((1,), (1,)), ((), ())),
                             preferred_element_type=jnp.float32)
        sq = S2 * S2
        ss0 = jnp.sum(jnp.where(m0w, sq, 0.0), axis=1, keepdims=True)
        ss1 = jnp.sum(jnp.where(m0w, 0.0, sq), axis=1, keepdims=True)
        d2 = jnp.where(m0w, ss0, ss1)
        Sn = S2 / jnp.maximum(jnp.sqrt(d2), 1e-12)
        tr0 = ts_ref[b0][None, :]
        tr1 = ts_ref[b1][None, :]
        tw0 = jnp.exp((jnp.max(tr0) - tr0) * _LN_DECAY)
        tw1 = jnp.exp((jnp.max(tr1) - tr1) * _LN_DECAY)
        A = jnp.concatenate([tw0, tw1], axis=0)
        Bv = jnp.concatenate(
            [jnp.concatenate([tw0, jnp.zeros((1, 64), jnp.float32)], axis=1),
             jnp.concatenate([jnp.zeros((1, 64), jnp.float32), tw1], axis=1)],
            axis=0)[:, :_W]
        T = lax.dot_general(A, Bv, (((0,), (0,)), ((), ())),
                            preferred_element_type=jnp.float32)
        iwA = jnp.concatenate([iw_ref[b0][None, :], iw_ref[b1][None, :]],
                              axis=0)
        onesB = jnp.concatenate([m0f, 1.0 - m0f], axis=0)
        Iw = lax.dot_general(iwA, onesB, (((0,), (0,)), ((), ())),
                             preferred_element_type=jnp.float32)
        acc = jnp.full((_S, _W), params_ref[5, 0], jnp.float32)
        for c in range(16):
            a_ = params_ref[0, c]
            b_ = params_ref[1, c]
            c_ = params_ref[2, c]
            d_ = params_ref[3, c]
            e_ = params_ref[4, c]
            h = jnp.maximum(a_ * Sn + b_ * T + c_ * Iw + d_, 0.0)
            acc = acc + e_ * h
        Z = jnp.concatenate(
            [1.0 / (1.0 + jnp.exp(-acc)), jnp.zeros((_S, 128 - _W))], axis=1)
        keys = jnp.where(triu & (Z > _THRESH), Z,
                         jnp.where(triu, -10000.0 - linf, _NEG))
        r0 = jnp.max(jnp.where(m0, keys, _NEGBIG), axis=1, keepdims=True)
        r1 = jnp.max(jnp.where(m0, _NEGBIG, keys), axis=1, keepdims=True)
        rmT = jnp.concatenate([r0, r1], axis=1).T
        rm = jnp.concatenate([rmT[0:1], zpad, rmT[1:2], zpad], axis=1)
        out_ref[p, 0:_S] = Z
        out_ref[p, _S:_S + 1] = rm
        out_ref[p, _S + 1:_ROWS] = rm


def _dense_weights(item_embeddings, timestamps, interaction_weights,
                   W1, b1, W2, b2, interpret=False):
    B = item_embeddings.shape[0]
    params = jnp.stack([W1[:, 0], W1[:, 1], W1[:, 2], b1, W2[0, :],
                        jnp.broadcast_to(b2, (16,))], axis=0)
    grid = (B // _BB,)
    return pl.pallas_call(
        _dense_body,
        grid=grid,
        in_specs=[
            pl.BlockSpec((6, 16), lambda i: (0, 0), memory_space=pltpu.SMEM),
            pl.BlockSpec((_BB, _S, _D), lambda i: (i, 0, 0)),
            pl.BlockSpec((_BB, _S), lambda i: (i, 0)),
            pl.BlockSpec((_BB, _S), lambda i: (i, 0)),
        ],
        out_specs=pl.BlockSpec((_BB // 2, _ROWS, 128), lambda i: (i, 0, 0)),
        out_shape=jax.ShapeDtypeStruct((B // 2, _ROWS, 128), jnp.float32),
        interpret=interpret,
    )(params, item_embeddings, timestamps, interaction_weights)


def _sc_topk_body(nw, pairs_per_w, pbase, w_hbm, src_hbm, dst_hbm, wout_hbm,
                  row_v, osrc_v, odst_v, ow_v):
    info = plsc.get_sparse_core_info()
    nc = info.num_cores
    wid = lax.axis_index("s") * nc + lax.axis_index("c")
    iota = lax.iota(jnp.int32, 16)
    lane0 = iota == 0

    def per_pair(pp, _):
        gp = wid * pairs_per_w + pp
        pltpu.sync_copy(w_hbm.at[gp], row_v)
        for h in (0, 1):
            boff = (2 * (gp + pbase) + h) * _S
            rmbase = _S * 128 + 64 * h
            obase = (pp * 2 + h) * _OUTW

            def kstep(k, _):
                best_v = jnp.full((16,), _NEGBIG, jnp.float32)
                best_i = jnp.zeros((16,), jnp.int32)
                for q in range(4):
                    v = row_v[pl.ds(rmbase + 16 * q, 16)]
                    ridx = iota + 16 * q
                    take = v > best_v
                    best_v = jnp.where(take, v, best_v)
                    best_i = jnp.where(take, ridx, best_i)
                m = jnp.max(best_v)
                istar = jnp.min(jnp.where(best_v == m, best_i, 9999))
                base = istar * 128 + 64 * h
                keyqs = []
                cand = jnp.full((16,), 9999, jnp.int32)
                for q in range(4):
                    wv = row_v[pl.ds(base + 16 * q, 16)]
                    gj = iota + 16 * q
                    triu = (gj > istar) & (gj < _S)
                    valid = triu & (wv > _THRESH)
                    fb = triu & (wv > -1.0e8) & (~valid)
                    linf = (istar * _S + gj).astype(jnp.float32)
                    keyq = jnp.where(valid, wv,
                                     jnp.where(fb, -10000.0 - linf, _NEG))
                    keyqs.append((keyq, gj))
                    cand = jnp.minimum(cand, jnp.where(keyq == m, gj, 9999))
                jstar = jnp.min(cand)
                g = base + jstar
                gvec = jnp.full((16,), g, jnp.int32)
                raw = plsc.load_gather(row_v, [gvec])
                kvec = jnp.full((16,), obase + k, jnp.int32)
                plsc.store_scatter(ow_v, [kvec], raw, mask=lane0)
                plsc.store_scatter(osrc_v, [kvec],
                                   jnp.full((16,), istar + boff, jnp.int32),
                                   mask=lane0)
                plsc.store_scatter(odst_v, [kvec],
                                   jnp.full((16,), jstar + boff, jnp.int32),
                                   mask=lane0)
                plsc.store_scatter(row_v, [gvec],
                                   jnp.full((16,), _POISON, jnp.float32),
                                   mask=lane0)
                m3 = jnp.full((16,), _NEGBIG, jnp.float32)
                for keyq, gj in keyqs:
                    m3 = jnp.maximum(m3, jnp.where(gj == jstar, _NEG, keyq))
                plsc.store_scatter(row_v,
                                   [jnp.full((16,), rmbase + istar, jnp.int32)],
                                   jnp.full((16,), jnp.max(m3), jnp.float32),
                                   mask=lane0)
                return 0

            lax.fori_loop(0, _K, kstep, 0)
        return 0

    lax.fori_loop(0, pairs_per_w, per_pair, 0)
    pltpu.sync_copy(osrc_v, src_hbm.at[wid])
    pltpu.sync_copy(odst_v, dst_hbm.at[wid])
    pltpu.sync_copy(ow_v, wout_hbm.at[wid])


def _sc_topk(w_flat, pbase=0):
    npair = w_flat.shape[0]
    info = plsc.get_sparse_core_info()
    nw = info.num_cores * info.num_subcores
    pairs_per_w = npair // nw
    owid = pairs_per_w * 2 * _OUTW
    mesh = plsc.VectorSubcoreMesh(core_axis_name="c", subcore_axis_name="s")
    fn = functools.partial(
        pl.kernel,
        mesh=mesh,
        compiler_params=pltpu.CompilerParams(needs_layout_passes=False),
        out_type=[
            jax.ShapeDtypeStruct((nw, owid), jnp.int32),
            jax.ShapeDtypeStruct((nw, owid), jnp.int32),
            jax.ShapeDtypeStruct((nw, owid), jnp.float32),
        ],
        scratch_types=[
            pltpu.VMEM((_PAIR,), jnp.float32),
            pltpu.VMEM((owid,), jnp.int32),
            pltpu.VMEM((owid,), jnp.int32),
            pltpu.VMEM((owid,), jnp.float32),
        ],
    )(functools.partial(_sc_topk_body, nw, pairs_per_w, pbase))
    return fn(w_flat)


def kernel(user_embeddings, item_embeddings, timestamps, interaction_weights,
           W1, b1, W2, b2):
    B = item_embeddings.shape[0]
    nsplit = 4
    Bh = B // nsplit
    parts = []
    for c in range(nsplit):
        sl = slice(c * Bh, (c + 1) * Bh)
        w = _dense_weights(item_embeddings[sl], timestamps[sl],
                           interaction_weights[sl], W1, b1, W2, b2)
        parts.append(_sc_topk(w.reshape(Bh // 2, _PAIR),
                              pbase=c * (Bh // 2)))
    src = jnp.concatenate([p[0] for p in parts], axis=0)
    dst = jnp.concatenate([p[1] for p in parts], axis=0)
    edge_w = jnp.concatenate([p[2] for p in parts], axis=0)
    src = src.reshape(B, _OUTW)
    dst = dst.reshape(B, _OUTW)
    edge_w = edge_w.reshape(B, _OUTW)
    edge_index = jnp.stack([src[:, :_K].reshape(-1), dst[:, :_K].reshape(-1)],
                           axis=0)
    edge_weights = edge_w[:, :_K].reshape(-1)
    return edge_index, edge_weights
